# baseline (speedup 1.0000x reference)
import functools
import math

import jax
import jax.numpy as jnp
from jax.experimental import pallas as pl
from jax.experimental.pallas import tpu as pltpu


def _fused_kernel(wq_ref, ke_ref, bq_ref, x_ref, mask_ref, idx_ref, loss_ref,
                  w2b_ref, cb_ref, ssum_ref, asum_ref, *, b, s, e, scale):
    bi = pl.program_id(0)
    sc = pl.program_id(1)
    nsc = pl.num_programs(1)

    @pl.when(jnp.logical_and(bi == 0, sc == 0))
    def _prep():
        w2 = jax.lax.dot_general(
            wq_ref[...], ke_ref[...], (((0,), (1,)), ((), ())),
            preferred_element_type=jnp.float32,
            precision=jax.lax.Precision.DEFAULT) * scale
        w2b_ref[...] = w2.astype(jnp.bfloat16)
        cb_ref[...] = jax.lax.dot_general(
            bq_ref[...], ke_ref[...], (((1,), (1,)), ((), ())),
            preferred_element_type=jnp.float32,
            precision=jax.lax.Precision.DEFAULT) * scale
        ssum_ref[...] = jnp.zeros_like(ssum_ref)
        asum_ref[...] = jnp.zeros_like(asum_ref)

    mask_ref[...] = jnp.ones_like(mask_ref)
    xb = x_ref[0].astype(jnp.bfloat16)
    logits = jax.lax.dot_general(
        xb, w2b_ref[...], (((1,), (0,)), ((), ())),
        preferred_element_type=jnp.float32) + cb_ref[...]
    mx = jnp.max(logits, axis=-1, keepdims=True)
    ex = jnp.exp(logits - mx)
    p = ex / jnp.sum(ex, axis=-1, keepdims=True)
    part_s = jnp.sum(p, axis=0, keepdims=True)
    rowmask = jax.lax.broadcasted_iota(jnp.int32, (b, 1), 0) == bi
    ssum_ref[...] += jnp.where(rowmask, part_s, 0.0)
    asum_ref[...] += jnp.sum(p * jnp.log(p + 1e-9), axis=(0, 1), keepdims=True)

    @pl.when(jnp.logical_and(bi == b - 1, sc == nsc - 1))
    def _finalize():
        rows = ssum_ref[...]
        lanes = jax.lax.broadcasted_iota(jnp.int32, (b, e), 1)
        idxmat = jnp.zeros((b, e), jnp.int32)
        for j in range(e):
            m = jnp.max(rows, axis=-1, keepdims=True)
            cand = jnp.where(rows >= m, lanes, e)
            sel = jnp.min(cand, axis=-1, keepdims=True)
            idxmat = jnp.where(lanes == j, sel, idxmat)
            rows = jnp.where(lanes == sel, -jnp.inf, rows)
        idx_ref[...] = idxmat
        lb_loss = math.log(1.0 / e) / e
        loss_ref[...] = 0.001 * lb_loss + 0.001 * asum_ref[...] / (b * s * e)


def kernel(x, Wq, bq, key_emb):
    b, s, d = x.shape
    e = key_emb.shape[0]
    scale = d ** (-0.5)
    T = 1024

    mask, idx, loss2 = pl.pallas_call(
        functools.partial(_fused_kernel, b=b, s=s, e=e, scale=scale),
        grid=(b, s // T),
        in_specs=[
            pl.BlockSpec((d, d), lambda bi, sc: (0, 0)),
            pl.BlockSpec((e, d), lambda bi, sc: (0, 0)),
            pl.BlockSpec((1, d), lambda bi, sc: (0, 0)),
            pl.BlockSpec((1, T, d), lambda bi, sc: (bi, sc, 0)),
        ],
        out_specs=(
            pl.BlockSpec((1, T, e), lambda bi, sc: (bi, sc, 0)),
            pl.BlockSpec((b, e), lambda bi, sc: (0, 0)),
            pl.BlockSpec((1, 1), lambda bi, sc: (0, 0)),
        ),
        out_shape=(
            jax.ShapeDtypeStruct((b, s, e), jnp.float32),
            jax.ShapeDtypeStruct((b, e), jnp.int32),
            jax.ShapeDtypeStruct((1, 1), jnp.float32),
        ),
        scratch_shapes=[
            pltpu.VMEM((d, e), jnp.bfloat16),
            pltpu.VMEM((1, e), jnp.float32),
            pltpu.VMEM((b, e), jnp.float32),
            pltpu.VMEM((1, 1), jnp.float32),
        ],
        compiler_params=pltpu.CompilerParams(
            dimension_semantics=("arbitrary", "arbitrary")),
    )(Wq, key_emb, bq.reshape(1, d), x)

    return mask, idx, loss2[0, 0]


# speedup vs baseline: 1.1397x; 1.1397x over previous
import functools
import math

import jax
import jax.numpy as jnp
from jax.experimental import pallas as pl
from jax.experimental.pallas import tpu as pltpu


def _fused_kernel(wq_ref, ke_ref, bq_ref, x_ref, idx_ref, loss_ref,
                  w2b_ref, cb_ref, ssum_ref, asum_ref, *, b, s, e, scale):
    bi = pl.program_id(0)
    sc = pl.program_id(1)
    nsc = pl.num_programs(1)

    @pl.when(jnp.logical_and(bi == 0, sc == 0))
    def _prep():
        w2 = jax.lax.dot_general(
            wq_ref[...], ke_ref[...], (((0,), (1,)), ((), ())),
            preferred_element_type=jnp.float32,
            precision=jax.lax.Precision.DEFAULT) * scale
        w2b_ref[...] = w2.astype(jnp.bfloat16)
        cb_ref[...] = jax.lax.dot_general(
            bq_ref[...], ke_ref[...], (((1,), (1,)), ((), ())),
            preferred_element_type=jnp.float32,
            precision=jax.lax.Precision.DEFAULT) * scale
        ssum_ref[...] = jnp.zeros_like(ssum_ref)
        asum_ref[...] = jnp.zeros_like(asum_ref)

    xb = x_ref[0].astype(jnp.bfloat16)
    logits = jax.lax.dot_general(
        xb, w2b_ref[...], (((1,), (0,)), ((), ())),
        preferred_element_type=jnp.float32) + cb_ref[...]
    mx = jnp.max(logits, axis=-1, keepdims=True)
    ex = jnp.exp(logits - mx)
    p = ex / jnp.sum(ex, axis=-1, keepdims=True)
    part_s = jnp.sum(p, axis=0, keepdims=True)
    rowmask = jax.lax.broadcasted_iota(jnp.int32, (b, 1), 0) == bi
    ssum_ref[...] += jnp.where(rowmask, part_s, 0.0)
    asum_ref[...] += jnp.sum(p * jnp.log(p + 1e-9), axis=(0, 1), keepdims=True)

    @pl.when(jnp.logical_and(bi == b - 1, sc == nsc - 1))
    def _finalize():
        rows = ssum_ref[...]
        lanes = jax.lax.broadcasted_iota(jnp.int32, (b, e), 1)
        idxmat = jnp.zeros((b, e), jnp.int32)
        for j in range(e):
            m = jnp.max(rows, axis=-1, keepdims=True)
            cand = jnp.where(rows >= m, lanes, e)
            sel = jnp.min(cand, axis=-1, keepdims=True)
            idxmat = jnp.where(lanes == j, sel, idxmat)
            rows = jnp.where(lanes == sel, -jnp.inf, rows)
        idx_ref[...] = idxmat
        lb_loss = math.log(1.0 / e) / e
        loss_ref[...] = 0.001 * lb_loss + 0.001 * asum_ref[...] / (b * s * e)


def kernel(x, Wq, bq, key_emb):
    b, s, d = x.shape
    e = key_emb.shape[0]
    scale = d ** (-0.5)
    T = 2048

    idx, loss2 = pl.pallas_call(
        functools.partial(_fused_kernel, b=b, s=s, e=e, scale=scale),
        grid=(b, s // T),
        in_specs=[
            pl.BlockSpec((d, d), lambda bi, sc: (0, 0)),
            pl.BlockSpec((e, d), lambda bi, sc: (0, 0)),
            pl.BlockSpec((1, d), lambda bi, sc: (0, 0)),
            pl.BlockSpec((1, T, d), lambda bi, sc: (bi, sc, 0)),
        ],
        out_specs=(
            pl.BlockSpec((b, e), lambda bi, sc: (0, 0)),
            pl.BlockSpec((1, 1), lambda bi, sc: (0, 0)),
        ),
        out_shape=(
            jax.ShapeDtypeStruct((b, e), jnp.int32),
            jax.ShapeDtypeStruct((1, 1), jnp.float32),
        ),
        scratch_shapes=[
            pltpu.VMEM((d, e), jnp.bfloat16),
            pltpu.VMEM((1, e), jnp.float32),
            pltpu.VMEM((b, e), jnp.float32),
            pltpu.VMEM((1, 1), jnp.float32),
        ],
        compiler_params=pltpu.CompilerParams(
            dimension_semantics=("arbitrary", "arbitrary")),
    )(Wq, key_emb, bq.reshape(1, d), x)

    mask = jnp.ones((b, s, e), jnp.float32)
    return mask, idx, loss2[0, 0]
